# baseline (speedup 1.0000x reference)
import functools

import jax
import jax.numpy as jnp
from jax import lax
from jax.experimental import pallas as pl
from jax.experimental.pallas import tpu as pltpu
from jax.experimental.pallas import tpu_sc as plsc

IDXW = 128
NBUF = 3
TBLK = 4096


def _make_format(vocab, d_model):
    half = TBLK // 2
    n_blk = (vocab + TBLK - 1) // TBLK
    rem = vocab - (n_blk - 1) * TBLK
    rhalf = rem // 2

    def tposed(x):
        eye = jnp.eye(d_model, dtype=jnp.float32)
        return jax.lax.dot_general(
            x, eye, (((0,), (0,)), ((), ())),
            preferred_element_type=jnp.float32,
        )

    def body(t_ref, w_ref):
        i = pl.program_id(0)
        blk = t_ref[...]

        @pl.when(i < n_blk - 1)
        def _full():
            w_ref[...] = jnp.concatenate(
                [tposed(blk[:, :half]), tposed(blk[:, half:])], axis=1
            )

        @pl.when(i == n_blk - 1)
        def _tail():
            w_ref[pl.ds(0, rhalf), 0:d_model] = tposed(blk[:, :rhalf])
            w_ref[pl.ds(0, rhalf), d_model:2 * d_model] = tposed(
                blk[:, rhalf:rem]
            )

    return pl.pallas_call(
        body,
        grid=(n_blk,),
        in_specs=[pl.BlockSpec((d_model, TBLK), lambda i: (0, i))],
        out_specs=pl.BlockSpec((half, 2 * d_model), lambda i: (i, 0)),
        out_shape=jax.ShapeDtypeStruct((vocab // 2, 2 * d_model), jnp.float32),
    )


def _make_gather(vocab, d_model, num_idx):
    info = plsc.get_sparse_core_info()
    nc, ns = info.num_cores, info.num_subcores
    nw = nc * ns
    per_w = num_idx // nw
    k = 4
    chunk = k * IDXW
    n_chunks = per_w // chunk
    idx_rows = per_w // IDXW
    assert per_w % chunk == 0 and n_chunks > NBUF

    mesh = plsc.VectorSubcoreMesh(core_axis_name="c", subcore_axis_name="s")

    @functools.partial(
        pl.kernel,
        mesh=mesh,
        compiler_params=pltpu.CompilerParams(use_tc_tiling_on_sc=False),
        out_type=jax.ShapeDtypeStruct((num_idx, 2 * d_model), jnp.float32),
        scratch_types=[
            pltpu.VMEM((idx_rows, IDXW), jnp.int32),
            [pltpu.VMEM((chunk, d_model), jnp.float32) for _ in range(NBUF)],
            [pltpu.SemaphoreType.DMA for _ in range(NBUF)],
            [pltpu.SemaphoreType.DMA for _ in range(NBUF)],
        ],
    )
    def gather_kernel(idx_hbm, table_hbm, out_hbm, idx_v, rows, sem_g, sem_st):
        wid = lax.axis_index("s") * nc + lax.axis_index("c")
        row0 = wid * idx_rows
        base = wid * per_w

        def fire_gathers(c, j):
            for q in range(k):
                pltpu.async_copy(
                    table_hbm.at[idx_v.at[c * k + q]],
                    rows[j].at[pl.ds(q * IDXW, IDXW)],
                    sem_g[j],
                )

        def wait_gathers(j):
            pltpu.make_async_copy(
                table_hbm.at[pl.ds(0, chunk)], rows[j], sem_g[j]
            ).wait()

        def fire_store(c, j):
            pltpu.async_copy(
                rows[j],
                out_hbm.at[pl.ds(base + c * chunk, chunk), pl.ds(0, d_model)],
                sem_st[j],
            )

        def wait_store(j):
            pltpu.make_async_copy(
                rows[j], out_hbm.at[pl.ds(0, chunk), pl.ds(0, d_model)],
                sem_st[j]
            ).wait()

        def steady(c, j):
            wait_store(j)
            fire_gathers(c, j)
            wait_gathers((j + 1) % NBUF)
            fire_store(c - 2, (j + 1) % NBUF)

        pltpu.sync_copy(idx_hbm.at[pl.ds(row0, idx_rows)], idx_v)
        for c in range(NBUF):
            fire_gathers(c, c)
        wait_gathers(0)
        fire_store(0, 0)

        groups = (n_chunks - NBUF) // NBUF
        tail0 = NBUF + groups * NBUF

        def body(i, carry):
            g = NBUF + i * NBUF
            for j in range(NBUF):
                steady(g + j, j)
            return carry

        lax.fori_loop(0, groups, body, 0)

        for c in range(tail0, n_chunks):
            steady(c, c % NBUF)
        for c in range(n_chunks - 2, n_chunks):
            wait_gathers(c % NBUF)
            fire_store(c, c % NBUF)
        for c in range(n_chunks - NBUF, n_chunks):
            wait_store(c % NBUF)

    return gather_kernel


def kernel(indices, table):
    b, s = indices.shape
    vocab, d_model = table.shape
    num_idx = b * s
    half = TBLK // 2
    n_blk = (vocab + TBLK - 1) // TBLK
    split = (n_blk - 1) * TBLK
    rhalf = (vocab - split) // 2

    lin = _make_format(vocab, d_model)(table.T).reshape(vocab, d_model)

    v = indices.astype(jnp.int32)
    vm = v % TBLK
    rho_main = (v - vm) + 2 * (vm % half) + vm // half
    r = v - split
    rho_tail = split + 2 * (r % rhalf) + r // rhalf
    rho = jnp.where(v < split, rho_main, rho_tail)

    idx2d = rho.reshape(num_idx // IDXW, IDXW)
    outp = _make_gather(vocab, d_model, num_idx)(idx2d, lin)
    return outp[:, :d_model].reshape(b, s, d_model)


# speedup vs baseline: 1.1120x; 1.1120x over previous
import functools

import jax
import jax.numpy as jnp
from jax import lax
from jax.experimental import pallas as pl
from jax.experimental.pallas import tpu as pltpu
from jax.experimental.pallas import tpu_sc as plsc

IDXW = 128
NBUF = 3
TBLK = 8192


def _make_format(vocab, d_model):
    half = TBLK // 2
    n_blk = (vocab + TBLK - 1) // TBLK
    rem = vocab - (n_blk - 1) * TBLK
    rhalf = rem // 2

    def tposed(x):
        eye = jnp.eye(d_model, dtype=jnp.float32)
        return jax.lax.dot_general(
            x, eye, (((0,), (0,)), ((), ())),
            preferred_element_type=jnp.float32,
        )

    def body(t_ref, w_ref):
        i = pl.program_id(0)
        blk = t_ref[...]

        @pl.when(i < n_blk - 1)
        def _full():
            w_ref[...] = jnp.concatenate(
                [tposed(blk[:, :half]), tposed(blk[:, half:])], axis=1
            )

        @pl.when(i == n_blk - 1)
        def _tail():
            w_ref[pl.ds(0, rhalf), 0:d_model] = tposed(blk[:, :rhalf])
            w_ref[pl.ds(0, rhalf), d_model:2 * d_model] = tposed(
                blk[:, rhalf:rem]
            )

    return pl.pallas_call(
        body,
        grid=(n_blk,),
        in_specs=[pl.BlockSpec((d_model, TBLK), lambda i: (0, i))],
        out_specs=pl.BlockSpec((half, 2 * d_model), lambda i: (i, 0)),
        out_shape=jax.ShapeDtypeStruct((vocab // 2, 2 * d_model), jnp.float32),
    )


def _make_gather(vocab, d_model, num_idx):
    info = plsc.get_sparse_core_info()
    nc, ns = info.num_cores, info.num_subcores
    nw = nc * ns
    per_w = num_idx // nw
    k = 4
    chunk = k * IDXW
    n_chunks = per_w // chunk
    idx_rows = per_w // IDXW
    assert per_w % chunk == 0 and n_chunks > NBUF

    mesh = plsc.VectorSubcoreMesh(core_axis_name="c", subcore_axis_name="s")

    @functools.partial(
        pl.kernel,
        mesh=mesh,
        compiler_params=pltpu.CompilerParams(use_tc_tiling_on_sc=False),
        out_type=jax.ShapeDtypeStruct((num_idx, 2 * d_model), jnp.float32),
        scratch_types=[
            pltpu.VMEM((idx_rows, IDXW), jnp.int32),
            [pltpu.VMEM((chunk, d_model), jnp.float32) for _ in range(NBUF)],
            [pltpu.SemaphoreType.DMA for _ in range(NBUF)],
            [pltpu.SemaphoreType.DMA for _ in range(NBUF)],
        ],
    )
    def gather_kernel(idx_hbm, table_hbm, out_hbm, idx_v, rows, sem_g, sem_st):
        wid = lax.axis_index("s") * nc + lax.axis_index("c")
        row0 = wid * idx_rows
        base = wid * per_w

        def fire_gathers(c, j):
            for q in range(k):
                pltpu.async_copy(
                    table_hbm.at[idx_v.at[c * k + q]],
                    rows[j].at[pl.ds(q * IDXW, IDXW)],
                    sem_g[j],
                )

        def wait_gathers(j):
            pltpu.make_async_copy(
                table_hbm.at[pl.ds(0, chunk)], rows[j], sem_g[j]
            ).wait()

        def fire_store(c, j):
            pltpu.async_copy(
                rows[j],
                out_hbm.at[pl.ds(base + c * chunk, chunk), pl.ds(0, d_model)],
                sem_st[j],
            )

        def wait_store(j):
            pltpu.make_async_copy(
                rows[j], out_hbm.at[pl.ds(0, chunk), pl.ds(0, d_model)],
                sem_st[j]
            ).wait()

        def steady(c, j):
            wait_store(j)
            fire_gathers(c, j)
            wait_gathers((j + 1) % NBUF)
            fire_store(c - 2, (j + 1) % NBUF)

        pltpu.sync_copy(idx_hbm.at[pl.ds(row0, idx_rows)], idx_v)
        for c in range(NBUF):
            fire_gathers(c, c)
        wait_gathers(0)
        fire_store(0, 0)

        groups = (n_chunks - NBUF) // NBUF
        tail0 = NBUF + groups * NBUF

        def body(i, carry):
            g = NBUF + i * NBUF
            for j in range(NBUF):
                steady(g + j, j)
            return carry

        lax.fori_loop(0, groups, body, 0)

        for c in range(tail0, n_chunks):
            steady(c, c % NBUF)
        for c in range(n_chunks - 2, n_chunks):
            wait_gathers(c % NBUF)
            fire_store(c, c % NBUF)
        for c in range(n_chunks - NBUF, n_chunks):
            wait_store(c % NBUF)

    return gather_kernel


def kernel(indices, table):
    b, s = indices.shape
    vocab, d_model = table.shape
    num_idx = b * s
    half = TBLK // 2
    n_blk = (vocab + TBLK - 1) // TBLK
    split = (n_blk - 1) * TBLK
    rhalf = (vocab - split) // 2

    lin = _make_format(vocab, d_model)(table.T).reshape(vocab, d_model)

    v = indices.astype(jnp.int32)
    vm = v % TBLK
    rho_main = (v - vm) + 2 * (vm % half) + vm // half
    r = v - split
    rho_tail = split + 2 * (r % rhalf) + r // rhalf
    rho = jnp.where(v < split, rho_main, rho_tail)

    idx2d = rho.reshape(num_idx // IDXW, IDXW)
    outp = _make_gather(vocab, d_model, num_idx)(idx2d, lin)
    return outp[:, :d_model].reshape(b, s, d_model)


# speedup vs baseline: 1.1813x; 1.0624x over previous
import functools

import jax
import jax.numpy as jnp
from jax import lax
from jax.experimental import pallas as pl
from jax.experimental.pallas import tpu as pltpu
from jax.experimental.pallas import tpu_sc as plsc

IDXW = 128
NBUF = 3
TBLK = 16384


def _make_format(vocab, d_model):
    half = TBLK // 2
    n_blk = (vocab + TBLK - 1) // TBLK
    rem = vocab - (n_blk - 1) * TBLK
    rhalf = rem // 2

    def tposed(x):
        eye = jnp.eye(d_model, dtype=jnp.float32)
        return jax.lax.dot_general(
            x, eye, (((0,), (0,)), ((), ())),
            preferred_element_type=jnp.float32,
        )

    def body(t_ref, w_ref):
        i = pl.program_id(0)
        blk = t_ref[...]

        @pl.when(i < n_blk - 1)
        def _full():
            w_ref[...] = jnp.concatenate(
                [tposed(blk[:, :half]), tposed(blk[:, half:])], axis=1
            )

        @pl.when(i == n_blk - 1)
        def _tail():
            w_ref[pl.ds(0, rhalf), 0:d_model] = tposed(blk[:, :rhalf])
            w_ref[pl.ds(0, rhalf), d_model:2 * d_model] = tposed(
                blk[:, rhalf:rem]
            )

    return pl.pallas_call(
        body,
        grid=(n_blk,),
        in_specs=[pl.BlockSpec((d_model, TBLK), lambda i: (0, i))],
        out_specs=pl.BlockSpec((half, 2 * d_model), lambda i: (i, 0)),
        out_shape=jax.ShapeDtypeStruct((vocab // 2, 2 * d_model), jnp.float32),
    )


def _make_gather(vocab, d_model, num_idx):
    info = plsc.get_sparse_core_info()
    nc, ns = info.num_cores, info.num_subcores
    nw = nc * ns
    per_w = num_idx // nw
    k = 4
    chunk = k * IDXW
    n_chunks = per_w // chunk
    idx_rows = per_w // IDXW
    assert per_w % chunk == 0 and n_chunks > NBUF

    mesh = plsc.VectorSubcoreMesh(core_axis_name="c", subcore_axis_name="s")

    @functools.partial(
        pl.kernel,
        mesh=mesh,
        compiler_params=pltpu.CompilerParams(use_tc_tiling_on_sc=False),
        out_type=jax.ShapeDtypeStruct((num_idx, 2 * d_model), jnp.float32),
        scratch_types=[
            pltpu.VMEM((idx_rows, IDXW), jnp.int32),
            [pltpu.VMEM((chunk, d_model), jnp.float32) for _ in range(NBUF)],
            [pltpu.SemaphoreType.DMA for _ in range(NBUF)],
            [pltpu.SemaphoreType.DMA for _ in range(NBUF)],
        ],
    )
    def gather_kernel(idx_hbm, table_hbm, out_hbm, idx_v, rows, sem_g, sem_st):
        wid = lax.axis_index("s") * nc + lax.axis_index("c")
        row0 = wid * idx_rows
        base = wid * per_w

        def fire_gathers(c, j):
            for q in range(k):
                pltpu.async_copy(
                    table_hbm.at[idx_v.at[c * k + q]],
                    rows[j].at[pl.ds(q * IDXW, IDXW)],
                    sem_g[j],
                )

        def wait_gathers(j):
            pltpu.make_async_copy(
                table_hbm.at[pl.ds(0, chunk)], rows[j], sem_g[j]
            ).wait()

        def fire_store(c, j):
            pltpu.async_copy(
                rows[j],
                out_hbm.at[pl.ds(base + c * chunk, chunk), pl.ds(0, d_model)],
                sem_st[j],
            )

        def wait_store(j):
            pltpu.make_async_copy(
                rows[j], out_hbm.at[pl.ds(0, chunk), pl.ds(0, d_model)],
                sem_st[j]
            ).wait()

        def steady(c, j):
            wait_store(j)
            fire_gathers(c, j)
            wait_gathers((j + 1) % NBUF)
            fire_store(c - 2, (j + 1) % NBUF)

        pltpu.sync_copy(idx_hbm.at[pl.ds(row0, idx_rows)], idx_v)
        for c in range(NBUF):
            fire_gathers(c, c)
        wait_gathers(0)
        fire_store(0, 0)

        groups = (n_chunks - NBUF) // NBUF
        tail0 = NBUF + groups * NBUF

        def body(i, carry):
            g = NBUF + i * NBUF
            for j in range(NBUF):
                steady(g + j, j)
            return carry

        lax.fori_loop(0, groups, body, 0)

        for c in range(tail0, n_chunks):
            steady(c, c % NBUF)
        for c in range(n_chunks - 2, n_chunks):
            wait_gathers(c % NBUF)
            fire_store(c, c % NBUF)
        for c in range(n_chunks - NBUF, n_chunks):
            wait_store(c % NBUF)

    return gather_kernel


def kernel(indices, table):
    b, s = indices.shape
    vocab, d_model = table.shape
    num_idx = b * s
    half = TBLK // 2
    n_blk = (vocab + TBLK - 1) // TBLK
    split = (n_blk - 1) * TBLK
    rhalf = (vocab - split) // 2

    lin = _make_format(vocab, d_model)(table.T).reshape(vocab, d_model)

    v = indices.astype(jnp.int32)
    vm = v % TBLK
    rho_main = (v - vm) + 2 * (vm % half) + vm // half
    r = v - split
    rho_tail = split + 2 * (r % rhalf) + r // rhalf
    rho = jnp.where(v < split, rho_main, rho_tail)

    idx2d = rho.reshape(num_idx // IDXW, IDXW)
    outp = _make_gather(vocab, d_model, num_idx)(idx2d, lin)
    return outp[:, :d_model].reshape(b, s, d_model)


# speedup vs baseline: 1.1990x; 1.0149x over previous
import functools

import jax
import jax.numpy as jnp
from jax import lax
from jax.experimental import pallas as pl
from jax.experimental.pallas import tpu as pltpu
from jax.experimental.pallas import tpu_sc as plsc

IDXW = 128
NBUF = 3
TBLK = 32768


def _make_format(vocab, d_model):
    half = TBLK // 2
    n_blk = (vocab + TBLK - 1) // TBLK
    rem = vocab - (n_blk - 1) * TBLK
    rhalf = rem // 2

    def tposed(x):
        eye = jnp.eye(d_model, dtype=jnp.float32)
        return jax.lax.dot_general(
            x, eye, (((0,), (0,)), ((), ())),
            preferred_element_type=jnp.float32,
        )

    def body(t_ref, w_ref):
        i = pl.program_id(0)
        blk = t_ref[...]

        @pl.when(i < n_blk - 1)
        def _full():
            w_ref[...] = jnp.concatenate(
                [tposed(blk[:, :half]), tposed(blk[:, half:])], axis=1
            )

        @pl.when(i == n_blk - 1)
        def _tail():
            w_ref[pl.ds(0, rhalf), 0:d_model] = tposed(blk[:, :rhalf])
            w_ref[pl.ds(0, rhalf), d_model:2 * d_model] = tposed(
                blk[:, rhalf:rem]
            )

    return pl.pallas_call(
        body,
        grid=(n_blk,),
        in_specs=[pl.BlockSpec((d_model, TBLK), lambda i: (0, i))],
        out_specs=pl.BlockSpec((half, 2 * d_model), lambda i: (i, 0)),
        out_shape=jax.ShapeDtypeStruct((vocab // 2, 2 * d_model), jnp.float32),
    )


def _make_gather(vocab, d_model, num_idx):
    info = plsc.get_sparse_core_info()
    nc, ns = info.num_cores, info.num_subcores
    nw = nc * ns
    per_w = num_idx // nw
    k = 4
    chunk = k * IDXW
    n_chunks = per_w // chunk
    idx_rows = per_w // IDXW
    assert per_w % chunk == 0 and n_chunks > NBUF

    mesh = plsc.VectorSubcoreMesh(core_axis_name="c", subcore_axis_name="s")

    @functools.partial(
        pl.kernel,
        mesh=mesh,
        compiler_params=pltpu.CompilerParams(use_tc_tiling_on_sc=False),
        out_type=jax.ShapeDtypeStruct((num_idx, 2 * d_model), jnp.float32),
        scratch_types=[
            pltpu.VMEM((idx_rows, IDXW), jnp.int32),
            [pltpu.VMEM((chunk, d_model), jnp.float32) for _ in range(NBUF)],
            [pltpu.SemaphoreType.DMA for _ in range(NBUF)],
            [pltpu.SemaphoreType.DMA for _ in range(NBUF)],
        ],
    )
    def gather_kernel(idx_hbm, table_hbm, out_hbm, idx_v, rows, sem_g, sem_st):
        wid = lax.axis_index("s") * nc + lax.axis_index("c")
        row0 = wid * idx_rows
        base = wid * per_w

        def fire_gathers(c, j):
            for q in range(k):
                pltpu.async_copy(
                    table_hbm.at[idx_v.at[c * k + q]],
                    rows[j].at[pl.ds(q * IDXW, IDXW)],
                    sem_g[j],
                )

        def wait_gathers(j):
            pltpu.make_async_copy(
                table_hbm.at[pl.ds(0, chunk)], rows[j], sem_g[j]
            ).wait()

        def fire_store(c, j):
            pltpu.async_copy(
                rows[j],
                out_hbm.at[pl.ds(base + c * chunk, chunk), pl.ds(0, d_model)],
                sem_st[j],
            )

        def wait_store(j):
            pltpu.make_async_copy(
                rows[j], out_hbm.at[pl.ds(0, chunk), pl.ds(0, d_model)],
                sem_st[j]
            ).wait()

        def steady(c, j):
            wait_store(j)
            fire_gathers(c, j)
            wait_gathers((j + 1) % NBUF)
            fire_store(c - 2, (j + 1) % NBUF)

        pltpu.sync_copy(idx_hbm.at[pl.ds(row0, idx_rows)], idx_v)
        for c in range(NBUF):
            fire_gathers(c, c)
        wait_gathers(0)
        fire_store(0, 0)

        groups = (n_chunks - NBUF) // NBUF
        tail0 = NBUF + groups * NBUF

        def body(i, carry):
            g = NBUF + i * NBUF
            for j in range(NBUF):
                steady(g + j, j)
            return carry

        lax.fori_loop(0, groups, body, 0)

        for c in range(tail0, n_chunks):
            steady(c, c % NBUF)
        for c in range(n_chunks - 2, n_chunks):
            wait_gathers(c % NBUF)
            fire_store(c, c % NBUF)
        for c in range(n_chunks - NBUF, n_chunks):
            wait_store(c % NBUF)

    return gather_kernel


def kernel(indices, table):
    b, s = indices.shape
    vocab, d_model = table.shape
    num_idx = b * s
    half = TBLK // 2
    n_blk = (vocab + TBLK - 1) // TBLK
    split = (n_blk - 1) * TBLK
    rhalf = (vocab - split) // 2

    lin = _make_format(vocab, d_model)(table.T).reshape(vocab, d_model)

    v = indices.astype(jnp.int32)
    vm = v % TBLK
    rho_main = (v - vm) + 2 * (vm % half) + vm // half
    r = v - split
    rho_tail = split + 2 * (r % rhalf) + r // rhalf
    rho = jnp.where(v < split, rho_main, rho_tail)

    idx2d = rho.reshape(num_idx // IDXW, IDXW)
    outp = _make_gather(vocab, d_model, num_idx)(idx2d, lin)
    return outp[:, :d_model].reshape(b, s, d_model)
